# baseline (speedup 1.0000x reference)
import functools

import jax
import jax.numpy as jnp
from jax import lax
from jax.experimental import pallas as pl
from jax.experimental.pallas import tpu as pltpu
from jax.experimental.pallas import tpu_sc as plsc

N = 100000
E = 3200000
G = 64
NP = 112000
SLICE = NP // 16
NW = 32
CH = 128
KCH = 8
SUP = CH * KCH
NCH = E // CH
NREAL = E // SUP
TPW = 98
NSLOT = 4
BN = 4000
GRID = N // BN
GRIDP = NP // BN

_CP = pltpu.CompilerParams(use_tc_tiling_on_sc=False,
                           needs_layout_passes=False)


def _hop_copy(src, dst, buf, src_base, dst_base, total):
    chunks = [(k * SUP, SUP) for k in range(total // SUP)]
    if total % SUP:
        chunks.append(((total // SUP) * SUP, total % SUP))
    for off, sz in chunks:
        bsl = pl.ds(0, sz)
        pltpu.sync_copy(src.at[pl.ds(src_base + off, sz)], buf.at[bsl])
        pltpu.sync_copy(buf.at[bsl], dst.at[pl.ds(dst_base + off, sz)])


def _make_edge_pass(D: int):
    mesh = plsc.VectorSubcoreMesh(core_axis_name="c", subcore_axis_name="s",
                                  num_cores=2, num_subcores=16)
    if D == 1:
        out_t = jax.ShapeDtypeStruct((2 * NP,), jnp.float32)
        rows_t = pltpu.VMEM((SUP,), jnp.float32)
        aggr_t = pltpu.VMEM_SHARED((NP,), jnp.float32)
        extra = [pltpu.VMEM((N,), jnp.float32)]
    else:
        out_t = jax.ShapeDtypeStruct((2 * NP, D), jnp.float32)
        rows_t = pltpu.VMEM((SUP, D), jnp.float32)
        aggr_t = pltpu.VMEM_SHARED((NP, D), jnp.float32)
        extra = []

    scratch = ([pltpu.VMEM((KCH, CH), jnp.int32)] * NSLOT
               + [pltpu.VMEM((KCH, CH), jnp.int32)] * NSLOT
               + [pltpu.VMEM((KCH, CH), jnp.float32)] * NSLOT
               + [rows_t] * NSLOT + [aggr_t] + extra
               + [pltpu.SemaphoreType.DMA] * (2 * NSLOT)
               + ([pltpu.SemaphoreType.DMA] * NSLOT if D != 1 else []))

    @functools.partial(pl.kernel, out_type=out_t, mesh=mesh,
                       scratch_types=scratch, compiler_params=_CP)
    def edge_pass(table, ei3, ewr, zi, zf, zeros, out, *sc):
        src_v = sc[0:NSLOT]
        dst_v = sc[NSLOT:2 * NSLOT]
        ew_v = sc[2 * NSLOT:3 * NSLOT]
        rows_v = sc[3 * NSLOT:4 * NSLOT]
        aggr_sh = sc[4 * NSLOT]
        nx = 4 * NSLOT + 1 + len(extra)
        tab = sc[4 * NSLOT + 1] if D == 1 else table
        lsem = sc[nx:nx + NSLOT]
        ssem = sc[nx + NSLOT:nx + 2 * NSLOT]
        gsem = sc[nx + 2 * NSLOT:]

        c = lax.axis_index("c")
        s = lax.axis_index("s")
        w = s * 2 + c
        base_g = w * TPW

        if D == 1:
            pltpu.sync_copy(table, tab)
        _hop_copy(zeros, aggr_sh, rows_v[0], s * SLICE, s * SLICE, SLICE)
        plsc.subcore_barrier()

        io = lax.iota(jnp.int32, 16)
        ro = io >> 3
        co = io & 7
        dn = lax.GatherDimensionNumbers(offset_dims=(),
                                        collapsed_slice_dims=(0,),
                                        start_index_map=(0,))

        def fire_stage(t, b):
            g = base_g + t
            r = pl.ds(g * KCH, KCH)

            @pl.when(g < NREAL)
            def _():
                pltpu.async_copy(ei3.at[0].at[r], src_v[b], lsem[b])
                pltpu.async_copy(ei3.at[1].at[r], dst_v[b], lsem[b])
                pltpu.async_copy(ewr.at[r], ew_v[b], lsem[b])

            @pl.when(g >= NREAL)
            def _():
                pltpu.async_copy(zi, src_v[b], lsem[b])
                pltpu.async_copy(zi, dst_v[b], lsem[b])
                pltpu.async_copy(zf, ew_v[b], lsem[b])

        def drain_stage(t, b):
            pltpu.make_async_copy(zi, src_v[b], lsem[b]).wait()
            pltpu.make_async_copy(zi, dst_v[b], lsem[b]).wait()
            pltpu.make_async_copy(zf, ew_v[b], lsem[b]).wait()

        def fire_gathers(b):
            for j in range(KCH):
                pltpu.async_copy(tab.at[src_v[b].at[j]],
                                 rows_v[b].at[pl.ds(j * CH, CH)], gsem[b])

        def drain_gathers(b):
            for j in range(KCH):
                pltpu.make_async_copy(tab.at[src_v[b].at[j]],
                                      rows_v[b].at[pl.ds(j * CH, CH)],
                                      gsem[b]).wait()

        def fire_scatters(b):
            for j in range(KCH):
                pltpu.async_copy(rows_v[b].at[pl.ds(j * CH, CH)],
                                 aggr_sh.at[dst_v[b].at[j]], ssem[b],
                                 add=True)

        def drain_scatters(b):
            for j in range(KCH):
                pltpu.make_async_copy(rows_v[b].at[pl.ds(j * CH, CH)],
                                      aggr_sh.at[dst_v[b].at[j]],
                                      ssem[b]).wait()

        if D == 1:
            def multiply(b):
                def mbody(m, carry):
                    rowv = jnp.full((16,), m >> 3, jnp.int32)
                    colv = io + (m % 8) * 16
                    sv = plsc.load_gather(src_v[b], [rowv, colv])
                    ew16 = plsc.load_gather(ew_v[b], [rowv, colv])
                    vals = plsc.load_gather(tab, [sv])
                    rows_v[b][pl.ds(pl.multiple_of(m * 16, 16), 16)] = (
                        vals * ew16)
                    return carry
                lax.fori_loop(0, SUP // 16, mbody, 0, unroll=8)
        else:
            def multiply(b):
                def mbody(m, carry):
                    rowv = jnp.full((16,), m >> 3, jnp.int32)
                    colv = io + (m % 8) * 16
                    ew16 = plsc.load_gather(ew_v[b], [rowv, colv])
                    bvec = ro + m * 16
                    for p in range(8):
                        ridx = bvec + 2 * p
                        vals = plsc.load_gather(rows_v[b], [ridx, co])
                        cidx = ro + 2 * p
                        sp = lax.gather(ew16, cidx[:, None], dn,
                                        slice_sizes=(1,),
                                        mode=lax.GatherScatterMode
                                        .PROMISE_IN_BOUNDS)
                        plsc.store_scatter(rows_v[b], [ridx, co], vals * sp)
                    return carry
                lax.fori_loop(0, SUP // 16, mbody, 0, unroll=4)

        def sub(t, b, fire_b, drain_a, fire_g):
            if drain_a:
                drain_scatters((b + 2) % NSLOT)
            if fire_b:
                fire_stage(t + 2, (b + 2) % NSLOT)
            if D == 1:
                drain_stage(t, b)
            elif fire_g:
                drain_stage(t + 1, (b + 1) % NSLOT)
                fire_gathers((b + 1) % NSLOT)
            if D != 1:
                drain_gathers(b)
            multiply(b)
            fire_scatters(b)

        fire_stage(0, 0)
        fire_stage(1, 1)
        if D != 1:
            drain_stage(0, 0)
            fire_gathers(0)
        sub(0, 0, True, False, True)
        sub(1, 1, True, False, True)
        sub(2, 2, True, True, True)
        sub(3, 3, True, True, True)

        def quad(q, carry):
            t0 = 4 + q * 4
            for k in range(4):
                sub(t0 + k, k, True, True, True)
            return carry

        lax.fori_loop(0, (TPW - 6) // 4, quad, 0)
        sub(TPW - 2, (TPW - 2) % NSLOT, False, True, True)
        sub(TPW - 1, (TPW - 1) % NSLOT, False, True, False)
        drain_scatters((TPW - 2) % NSLOT)
        drain_scatters((TPW - 1) % NSLOT)

        plsc.subcore_barrier()
        _hop_copy(aggr_sh, out, rows_v[0], s * SLICE, c * NP + s * SLICE,
                  SLICE)

    return edge_pass


_edge_pass_1 = _make_edge_pass(1)
_edge_pass_8 = _make_edge_pass(8)


def _b1_body(xb, a0, a1, bb, W1r, R1r, b1r, msr,
             mean_out, var_out, S1, S2, CNT):
    i = pl.program_id(0)

    @pl.when(i == 0)
    def _():
        S1[...] = jnp.zeros_like(S1)
        S2[...] = jnp.zeros_like(S2)
        CNT[...] = jnp.zeros_like(CNT)

    aggr = a0[...] + a1[...]
    h = aggr * W1r[...] + xb[...] * R1r[...] + b1r[...]
    h = jnp.maximum(h, 0.0)
    oh = (bb[...] == lax.broadcasted_iota(jnp.int32, (1, G), 1)
          ).astype(jnp.float32)
    dn = (((0,), (0,)), ((), ()))
    S1[...] += lax.dot_general(oh, h, dn, preferred_element_type=jnp.float32)
    S2[...] += lax.dot_general(oh, h * h, dn,
                               preferred_element_type=jnp.float32)
    CNT[...] += lax.dot_general(oh, jnp.ones((BN, 1), jnp.float32), dn,
                                preferred_element_type=jnp.float32)

    @pl.when(i == GRIDP - 1)
    def _():
        inv = 1.0 / jnp.maximum(CNT[...], 1.0)
        mean = S1[...] * inv
        msq = S2[...] * inv
        ms = msr[...]
        var = msq - mean * mean * (ms * (2.0 - ms))
        mean_out[...] = mean
        var_out[...] = jnp.maximum(var, 0.0)


def _b2_body(xb, a0, a1, bb, W1r, R1r, b1r, mean, var, msr, wr, br, hn_out):
    aggr = a0[...] + a1[...]
    h = aggr * W1r[...] + xb[...] * R1r[...] + b1r[...]
    h = jnp.maximum(h, 0.0)
    oh = (bb[...] == lax.broadcasted_iota(jnp.int32, (1, G), 1)
          ).astype(jnp.float32)
    mb = jnp.dot(oh, mean[...] * msr[...], preferred_element_type=jnp.float32)
    vb = jnp.dot(oh, var[...], preferred_element_type=jnp.float32)
    sub = h - mb
    hn_out[...] = wr[...] * sub / jnp.sqrt(vb + 1e-5) + br[...]


def _d_body(hn, q0, q1, Wmu, Rmu, Wlv, Rlv, bmur, blvr, mu_out, lv_out):
    ag = q0[...] + q1[...]
    h = hn[...]
    mu_out[...] = (jnp.dot(ag, Wmu[...], preferred_element_type=jnp.float32)
                   + jnp.dot(h, Rmu[...], preferred_element_type=jnp.float32)
                   + bmur[...])
    lv_out[...] = (jnp.dot(ag, Wlv[...], preferred_element_type=jnp.float32)
                   + jnp.dot(h, Rlv[...], preferred_element_type=jnp.float32)
                   + blvr[...])


def _full(shape):
    return pl.BlockSpec(shape, lambda i: (0, 0))


def _rows(d):
    return pl.BlockSpec((BN, d), lambda i: (i, 0))


def kernel(x, edge_index, edge_weight, batch, W1, b1, R1, gn_w, gn_b, gn_ms,
           Wmu, bmu, Rmu, Wlv, blv, Rlv):
    f32 = jnp.float32
    i32 = jnp.int32
    ei3 = edge_index.reshape(2, NCH, CH)
    ewr = edge_weight.reshape(NCH, CH)
    zi = jnp.zeros((KCH, CH), i32)
    zf = jnp.zeros((KCH, CH), f32)
    zeros1 = jnp.zeros((NP,), f32)
    zeros8 = jnp.zeros((NP, 8), f32)

    out1 = _edge_pass_1(x.reshape(N), ei3, ewr, zi, zf, zeros1)
    a0 = out1[:NP].reshape(NP, 1)
    a1 = out1[NP:].reshape(NP, 1)

    xp = jnp.concatenate([x, jnp.zeros((NP - N, 1), f32)]).reshape(NP, 1)
    bbp = jnp.concatenate([batch, jnp.full((NP - N,), G, i32)]
                          ).reshape(NP, 1)
    mean, var = pl.pallas_call(
        _b1_body,
        grid=(GRIDP,),
        in_specs=[_rows(1), _rows(1), _rows(1), _rows(1),
                  _full((1, 8)), _full((1, 8)), _full((1, 8)), _full((1, 8))],
        out_specs=[_full((G, 8)), _full((G, 8))],
        out_shape=[jax.ShapeDtypeStruct((G, 8), f32),
                   jax.ShapeDtypeStruct((G, 8), f32)],
        scratch_shapes=[pltpu.VMEM((G, 8), f32), pltpu.VMEM((G, 8), f32),
                        pltpu.VMEM((G, 1), f32)],
        compiler_params=pltpu.CompilerParams(
            dimension_semantics=("arbitrary",)),
    )(xp, a0, a1, bbp, W1.reshape(1, 8), R1.reshape(1, 8), b1.reshape(1, 8),
      gn_ms.reshape(1, 8))

    hn = pl.pallas_call(
        _b2_body,
        grid=(GRIDP,),
        in_specs=[_rows(1), _rows(1), _rows(1), _rows(1),
                  _full((1, 8)), _full((1, 8)), _full((1, 8)),
                  _full((G, 8)), _full((G, 8)),
                  _full((1, 8)), _full((1, 8)), _full((1, 8))],
        out_specs=_rows(8),
        out_shape=jax.ShapeDtypeStruct((NP, 8), f32),
    )(xp, a0, a1, bbp, W1.reshape(1, 8), R1.reshape(1, 8), b1.reshape(1, 8),
      mean, var, gn_ms.reshape(1, 8), gn_w.reshape(1, 8),
      gn_b.reshape(1, 8))

    out8 = _edge_pass_8(hn, ei3, ewr, zi, zf, zeros8)

    mu, logvar = pl.pallas_call(
        _d_body,
        grid=(GRID,),
        in_specs=[_rows(8),
                  pl.BlockSpec((BN, 8), lambda i: (i, 0)),
                  pl.BlockSpec((BN, 8), lambda i: (i + GRIDP, 0)),
                  _full((8, 16)), _full((8, 16)), _full((8, 16)),
                  _full((8, 16)), _full((1, 16)), _full((1, 16))],
        out_specs=[_rows(16), _rows(16)],
        out_shape=[jax.ShapeDtypeStruct((N, 16), f32),
                   jax.ShapeDtypeStruct((N, 16), f32)],
    )(hn, out8, out8, Wmu, Rmu, Wlv, Rlv, bmu.reshape(1, 16),
      blv.reshape(1, 16))

    return (mu, logvar)


# speedup vs baseline: 1.0047x; 1.0047x over previous
import functools

import jax
import jax.numpy as jnp
from jax import lax
from jax.experimental import pallas as pl
from jax.experimental.pallas import tpu as pltpu
from jax.experimental.pallas import tpu_sc as plsc

N = 100000
E = 3200000
G = 64
NP = 112000
SLICE = NP // 16
NW = 32
CH = 128
KCH = 8
SUP = CH * KCH
NCH = E // CH
NREAL = E // SUP
TPW = 98
NSLOT = 4
BN = 4000
GRID = N // BN
GRIDP = NP // BN

_CP = pltpu.CompilerParams(use_tc_tiling_on_sc=False,
                           needs_layout_passes=False)


def _hop_copy(src, dst, buf, src_base, dst_base, total):
    chunks = [(k * SUP, SUP) for k in range(total // SUP)]
    if total % SUP:
        chunks.append(((total // SUP) * SUP, total % SUP))
    for off, sz in chunks:
        bsl = pl.ds(0, sz)
        pltpu.sync_copy(src.at[pl.ds(src_base + off, sz)], buf.at[bsl])
        pltpu.sync_copy(buf.at[bsl], dst.at[pl.ds(dst_base + off, sz)])


def _make_edge_pass(D: int):
    mesh = plsc.VectorSubcoreMesh(core_axis_name="c", subcore_axis_name="s",
                                  num_cores=2, num_subcores=16)
    if D == 1:
        out_t = jax.ShapeDtypeStruct((2 * NP,), jnp.float32)
        rows_t = pltpu.VMEM((SUP,), jnp.float32)
        aggr_t = pltpu.VMEM_SHARED((NP,), jnp.float32)
        extra = [pltpu.VMEM((N,), jnp.float32)]
    else:
        out_t = jax.ShapeDtypeStruct((2 * NP, D), jnp.float32)
        rows_t = pltpu.VMEM((SUP, D), jnp.float32)
        aggr_t = pltpu.VMEM_SHARED((NP, D), jnp.float32)
        extra = []

    scratch = ([pltpu.VMEM((KCH, CH), jnp.int32)] * NSLOT
               + [pltpu.VMEM((KCH, CH), jnp.int32)] * NSLOT
               + [pltpu.VMEM((KCH, CH), jnp.float32)] * NSLOT
               + [rows_t] * NSLOT + [aggr_t] + extra
               + [pltpu.SemaphoreType.DMA] * (2 * NSLOT)
               + ([pltpu.SemaphoreType.DMA] * NSLOT if D != 1 else []))

    @functools.partial(pl.kernel, out_type=out_t, mesh=mesh,
                       scratch_types=scratch, compiler_params=_CP)
    def edge_pass(table, ei3, ewr, zi, zf, zeros, out, *sc):
        src_v = sc[0:NSLOT]
        dst_v = sc[NSLOT:2 * NSLOT]
        ew_v = sc[2 * NSLOT:3 * NSLOT]
        rows_v = sc[3 * NSLOT:4 * NSLOT]
        aggr_sh = sc[4 * NSLOT]
        nx = 4 * NSLOT + 1 + len(extra)
        tab = sc[4 * NSLOT + 1] if D == 1 else table
        lsem = sc[nx:nx + NSLOT]
        ssem = sc[nx + NSLOT:nx + 2 * NSLOT]
        gsem = sc[nx + 2 * NSLOT:]

        c = lax.axis_index("c")
        s = lax.axis_index("s")
        w = s * 2 + c
        base_g = w * TPW

        if D == 1:
            pltpu.sync_copy(table, tab)
        _hop_copy(zeros, aggr_sh, rows_v[0], s * SLICE, s * SLICE, SLICE)
        plsc.subcore_barrier()

        io = lax.iota(jnp.int32, 16)
        ro = io >> 3
        co = io & 7
        dn = lax.GatherDimensionNumbers(offset_dims=(),
                                        collapsed_slice_dims=(0,),
                                        start_index_map=(0,))

        def fire_stage(t, b):
            g = base_g + t
            r = pl.ds(g * KCH, KCH)

            @pl.when(g < NREAL)
            def _():
                pltpu.async_copy(ei3.at[0].at[r], src_v[b], lsem[b])
                pltpu.async_copy(ei3.at[1].at[r], dst_v[b], lsem[b])
                pltpu.async_copy(ewr.at[r], ew_v[b], lsem[b])

            @pl.when(g >= NREAL)
            def _():
                pltpu.async_copy(zi, src_v[b], lsem[b])
                pltpu.async_copy(zi, dst_v[b], lsem[b])
                pltpu.async_copy(zf, ew_v[b], lsem[b])

        def drain_stage(t, b):
            pltpu.make_async_copy(zi, src_v[b], lsem[b]).wait()
            pltpu.make_async_copy(zi, dst_v[b], lsem[b]).wait()
            pltpu.make_async_copy(zf, ew_v[b], lsem[b]).wait()

        def fire_gathers(b):
            for j in range(KCH):
                pltpu.async_copy(tab.at[src_v[b].at[j]],
                                 rows_v[b].at[pl.ds(j * CH, CH)], gsem[b])

        def drain_gathers(b):
            for j in range(KCH):
                pltpu.make_async_copy(tab.at[src_v[b].at[j]],
                                      rows_v[b].at[pl.ds(j * CH, CH)],
                                      gsem[b]).wait()

        def fire_scatters(b):
            for j in range(KCH):
                pltpu.async_copy(rows_v[b].at[pl.ds(j * CH, CH)],
                                 aggr_sh.at[dst_v[b].at[j]], ssem[b],
                                 add=True)

        def drain_scatters(b):
            for j in range(KCH):
                pltpu.make_async_copy(rows_v[b].at[pl.ds(j * CH, CH)],
                                      aggr_sh.at[dst_v[b].at[j]],
                                      ssem[b]).wait()

        if D == 1:
            def multiply(b):
                def mbody(m, carry):
                    rowv = jnp.full((16,), m >> 3, jnp.int32)
                    colv = io + (m % 8) * 16
                    sv = plsc.load_gather(src_v[b], [rowv, colv])
                    ew16 = plsc.load_gather(ew_v[b], [rowv, colv])
                    vals = plsc.load_gather(tab, [sv])
                    rows_v[b][pl.ds(pl.multiple_of(m * 16, 16), 16)] = (
                        vals * ew16)
                    return carry
                lax.fori_loop(0, SUP // 16, mbody, 0, unroll=4)
        else:
            def multiply(b):
                def mbody(m, carry):
                    rowv = jnp.full((16,), m >> 3, jnp.int32)
                    colv = io + (m % 8) * 16
                    ew16 = plsc.load_gather(ew_v[b], [rowv, colv])
                    bvec = ro + m * 16
                    for p in range(8):
                        ridx = bvec + 2 * p
                        vals = plsc.load_gather(rows_v[b], [ridx, co])
                        cidx = ro + 2 * p
                        sp = lax.gather(ew16, cidx[:, None], dn,
                                        slice_sizes=(1,),
                                        mode=lax.GatherScatterMode
                                        .PROMISE_IN_BOUNDS)
                        plsc.store_scatter(rows_v[b], [ridx, co], vals * sp)
                    return carry
                lax.fori_loop(0, SUP // 16, mbody, 0, unroll=2)

        def sub(t, b, fire_b, drain_a, fire_g):
            if drain_a:
                drain_scatters((b + 2) % NSLOT)
            if fire_b:
                fire_stage(t + 2, (b + 2) % NSLOT)
            if D == 1:
                drain_stage(t, b)
            elif fire_g:
                drain_stage(t + 1, (b + 1) % NSLOT)
                fire_gathers((b + 1) % NSLOT)
            if D != 1:
                drain_gathers(b)
            multiply(b)
            fire_scatters(b)

        fire_stage(0, 0)
        fire_stage(1, 1)
        if D != 1:
            drain_stage(0, 0)
            fire_gathers(0)
        sub(0, 0, True, False, True)
        sub(1, 1, True, False, True)
        sub(2, 2, True, True, True)
        sub(3, 3, True, True, True)

        def quad(q, carry):
            t0 = 4 + q * 4
            for k in range(4):
                sub(t0 + k, k, True, True, True)
            return carry

        lax.fori_loop(0, (TPW - 6) // 4, quad, 0)
        sub(TPW - 2, (TPW - 2) % NSLOT, False, True, True)
        sub(TPW - 1, (TPW - 1) % NSLOT, False, True, False)
        drain_scatters((TPW - 2) % NSLOT)
        drain_scatters((TPW - 1) % NSLOT)

        plsc.subcore_barrier()
        _hop_copy(aggr_sh, out, rows_v[0], s * SLICE, c * NP + s * SLICE,
                  SLICE)

    return edge_pass


_edge_pass_1 = _make_edge_pass(1)
_edge_pass_8 = _make_edge_pass(8)


def _b1_body(xb, a0, a1, bb, W1r, R1r, b1r, msr,
             mean_out, var_out, S1, S2, CNT):
    i = pl.program_id(0)

    @pl.when(i == 0)
    def _():
        S1[...] = jnp.zeros_like(S1)
        S2[...] = jnp.zeros_like(S2)
        CNT[...] = jnp.zeros_like(CNT)

    aggr = a0[...] + a1[...]
    h = aggr * W1r[...] + xb[...] * R1r[...] + b1r[...]
    h = jnp.maximum(h, 0.0)
    oh = (bb[...] == lax.broadcasted_iota(jnp.int32, (1, G), 1)
          ).astype(jnp.float32)
    dn = (((0,), (0,)), ((), ()))
    S1[...] += lax.dot_general(oh, h, dn, preferred_element_type=jnp.float32)
    S2[...] += lax.dot_general(oh, h * h, dn,
                               preferred_element_type=jnp.float32)
    CNT[...] += lax.dot_general(oh, jnp.ones((BN, 1), jnp.float32), dn,
                                preferred_element_type=jnp.float32)

    @pl.when(i == GRIDP - 1)
    def _():
        inv = 1.0 / jnp.maximum(CNT[...], 1.0)
        mean = S1[...] * inv
        msq = S2[...] * inv
        ms = msr[...]
        var = msq - mean * mean * (ms * (2.0 - ms))
        mean_out[...] = mean
        var_out[...] = jnp.maximum(var, 0.0)


def _b2_body(xb, a0, a1, bb, W1r, R1r, b1r, mean, var, msr, wr, br, hn_out):
    aggr = a0[...] + a1[...]
    h = aggr * W1r[...] + xb[...] * R1r[...] + b1r[...]
    h = jnp.maximum(h, 0.0)
    oh = (bb[...] == lax.broadcasted_iota(jnp.int32, (1, G), 1)
          ).astype(jnp.float32)
    mb = jnp.dot(oh, mean[...] * msr[...], preferred_element_type=jnp.float32)
    vb = jnp.dot(oh, var[...], preferred_element_type=jnp.float32)
    sub = h - mb
    hn_out[...] = wr[...] * sub / jnp.sqrt(vb + 1e-5) + br[...]


def _d_body(hn, q0, q1, Wmu, Rmu, Wlv, Rlv, bmur, blvr, mu_out, lv_out):
    ag = q0[...] + q1[...]
    h = hn[...]
    mu_out[...] = (jnp.dot(ag, Wmu[...], preferred_element_type=jnp.float32)
                   + jnp.dot(h, Rmu[...], preferred_element_type=jnp.float32)
                   + bmur[...])
    lv_out[...] = (jnp.dot(ag, Wlv[...], preferred_element_type=jnp.float32)
                   + jnp.dot(h, Rlv[...], preferred_element_type=jnp.float32)
                   + blvr[...])


def _full(shape):
    return pl.BlockSpec(shape, lambda i: (0, 0))


def _rows(d):
    return pl.BlockSpec((BN, d), lambda i: (i, 0))


def kernel(x, edge_index, edge_weight, batch, W1, b1, R1, gn_w, gn_b, gn_ms,
           Wmu, bmu, Rmu, Wlv, blv, Rlv):
    f32 = jnp.float32
    i32 = jnp.int32
    ei3 = edge_index.reshape(2, NCH, CH)
    ewr = edge_weight.reshape(NCH, CH)
    zi = jnp.zeros((KCH, CH), i32)
    zf = jnp.zeros((KCH, CH), f32)
    zeros1 = jnp.zeros((NP,), f32)
    zeros8 = jnp.zeros((NP, 8), f32)

    out1 = _edge_pass_1(x.reshape(N), ei3, ewr, zi, zf, zeros1)
    a0 = out1[:NP].reshape(NP, 1)
    a1 = out1[NP:].reshape(NP, 1)

    xp = jnp.concatenate([x, jnp.zeros((NP - N, 1), f32)]).reshape(NP, 1)
    bbp = jnp.concatenate([batch, jnp.full((NP - N,), G, i32)]
                          ).reshape(NP, 1)
    mean, var = pl.pallas_call(
        _b1_body,
        grid=(GRIDP,),
        in_specs=[_rows(1), _rows(1), _rows(1), _rows(1),
                  _full((1, 8)), _full((1, 8)), _full((1, 8)), _full((1, 8))],
        out_specs=[_full((G, 8)), _full((G, 8))],
        out_shape=[jax.ShapeDtypeStruct((G, 8), f32),
                   jax.ShapeDtypeStruct((G, 8), f32)],
        scratch_shapes=[pltpu.VMEM((G, 8), f32), pltpu.VMEM((G, 8), f32),
                        pltpu.VMEM((G, 1), f32)],
        compiler_params=pltpu.CompilerParams(
            dimension_semantics=("arbitrary",)),
    )(xp, a0, a1, bbp, W1.reshape(1, 8), R1.reshape(1, 8), b1.reshape(1, 8),
      gn_ms.reshape(1, 8))

    hn = pl.pallas_call(
        _b2_body,
        grid=(GRIDP,),
        in_specs=[_rows(1), _rows(1), _rows(1), _rows(1),
                  _full((1, 8)), _full((1, 8)), _full((1, 8)),
                  _full((G, 8)), _full((G, 8)),
                  _full((1, 8)), _full((1, 8)), _full((1, 8))],
        out_specs=_rows(8),
        out_shape=jax.ShapeDtypeStruct((NP, 8), f32),
    )(xp, a0, a1, bbp, W1.reshape(1, 8), R1.reshape(1, 8), b1.reshape(1, 8),
      mean, var, gn_ms.reshape(1, 8), gn_w.reshape(1, 8),
      gn_b.reshape(1, 8))

    out8 = _edge_pass_8(hn, ei3, ewr, zi, zf, zeros8)
    mu, logvar = pl.pallas_call(
        _d_body,
        grid=(GRID,),
        in_specs=[_rows(8),
                  pl.BlockSpec((BN, 8), lambda i: (i, 0)),
                  pl.BlockSpec((BN, 8), lambda i: (i + GRIDP, 0)),
                  _full((8, 16)), _full((8, 16)), _full((8, 16)),
                  _full((8, 16)), _full((1, 16)), _full((1, 16))],
        out_specs=[_rows(16), _rows(16)],
        out_shape=[jax.ShapeDtypeStruct((N, 16), f32),
                   jax.ShapeDtypeStruct((N, 16), f32)],
    )(hn, out8, out8, Wmu, Rmu, Wlv, Rlv, bmu.reshape(1, 16),
      blv.reshape(1, 16))

    return (mu, logvar)
